# baseline (speedup 1.0000x reference)
import functools

import jax
import jax.numpy as jnp
from jax import lax
from jax.experimental import pallas as pl
from jax.experimental.pallas import tpu as pltpu
from jax.experimental.pallas import tpu_sc as plsc

VOCAB = 1000000
DIM = 64
BASE = 10000.0
BATCH = 1024
SEQ = 200
LANES = 16

NUM_CORES = 2
NUM_SUBCORES = 16
NW = NUM_CORES * NUM_SUBCORES
ROWS_TOTAL = BATCH * SEQ
ROWS_PER_W = ROWS_TOTAL // NW
SEQS_PER_W = ROWS_PER_W // SEQ


def _pe_block(seq_len):
    theta_ids = jnp.arange(0, DIM, 2)
    theta = 1.0 / (BASE ** (theta_ids.astype(jnp.float32) / DIM))
    pe = jnp.zeros((DIM,), dtype=jnp.float32)
    pe = pe.at[theta_ids].set(theta)
    pe = pe.at[theta_ids + 1].set(theta)
    position_ids = jnp.arange(0, seq_len).astype(jnp.float32)
    out = jnp.outer(position_ids, pe)
    return jnp.sin(out)


def _sc_body(ids_hbm, table_hbm, pe_hbm, out_hbm, idx_v, pe_v, buf, sem_g):
    wid = lax.axis_index("s") * NUM_CORES + lax.axis_index("c")
    base_row = wid * ROWS_PER_W

    pltpu.sync_copy(ids_hbm.at[pl.ds(base_row, ROWS_PER_W)], idx_v)
    pltpu.sync_copy(pe_hbm, pe_v)

    def per_seq(q, _):
        row0 = q * SEQ
        g1 = pltpu.async_copy(
            table_hbm.at[idx_v.at[pl.ds(row0, 128)]],
            buf.at[pl.ds(0, 128)], sem_g)
        g2 = pltpu.async_copy(
            table_hbm.at[idx_v.at[pl.ds(row0 + 128, SEQ - 128)]],
            buf.at[pl.ds(128, SEQ - 128)], sem_g)
        g1.wait()
        g2.wait()

        def add_row(i, _):
            for qq in range(DIM // LANES):
                sl = pl.ds(qq * LANES, LANES)
                buf[i, sl] = buf[i, sl] + pe_v[i, sl]
            return _

        lax.fori_loop(0, SEQ, add_row, None)

        pltpu.sync_copy(buf, out_hbm.at[pl.ds(base_row + row0, SEQ)])
        return _

    lax.fori_loop(0, SEQS_PER_W, per_seq, None)


@jax.jit
def _run(ids_flat, table, pe):
    mesh = plsc.VectorSubcoreMesh(core_axis_name="c", subcore_axis_name="s")
    f = pl.kernel(
        _sc_body,
        out_type=jax.ShapeDtypeStruct((ROWS_TOTAL, DIM), jnp.float32),
        mesh=mesh,
        scratch_types=[
            pltpu.VMEM((ROWS_PER_W,), jnp.int32),
            pltpu.VMEM((SEQ, DIM), jnp.float32),
            pltpu.VMEM((SEQ, DIM), jnp.float32),
            pltpu.SemaphoreType.DMA,
        ],
        compiler_params=pltpu.CompilerParams(use_tc_tiling_on_sc=False),
    )
    return f(ids_flat, table, pe)


def kernel(input_ids, emb_table):
    ids_flat = input_ids.reshape(-1).astype(jnp.int32)
    pe = _pe_block(SEQ)
    out = _run(ids_flat, emb_table, pe)
    return out.reshape(BATCH, SEQ, DIM)
